# Initial kernel scaffold; baseline (speedup 1.0000x reference)
import jax
import jax.numpy as jnp
from jax.experimental import pallas as pl


def kernel(r_hist, distances, z_vals):
    raise NotImplementedError("write your pallas kernel here")


# speedup vs baseline: 46.9624x; 46.9624x over previous
import jax
import jax.numpy as jnp
from jax.experimental import pallas as pl

_RBLK = 128


def _evolve_block(hist_ref, d_ref, z_ref, out_ref):
    z = z_ref[...]
    d = d_ref[...]
    key = z[:, :, None] - d[:, None, :]
    pos = jnp.where(key < 0.0, jnp.float32(10.0), key)
    neg = jnp.where(key > 0.0, jnp.float32(-10.0), key)
    vals = jnp.min(pos, axis=-1)
    idx0 = jnp.argmin(pos, axis=-1).astype(jnp.int32)
    idx1 = jnp.argmax(neg, axis=-1).astype(jnp.int32)
    c0 = [None] * 3
    c1 = [None] * 3
    for c in range(3):
        hc = hist_ref[c]
        c0[c] = jnp.take_along_axis(hc, idx0, axis=-1)
        c1[c] = jnp.take_along_axis(hc, idx1, axis=-1)
    m = [(c1[c] - c0[c]) / z for c in range(3)]
    norm = jnp.sqrt(m[0] * m[0] + m[1] * m[1] + m[2] * m[2])
    for c in range(3):
        out_ref[c] = c0[c] + vals * (m[c] / norm)


def kernel(r_hist, distances, z_vals):
    B, T = distances.shape
    S = z_vals.shape[1]
    hist_t = jnp.transpose(r_hist, (2, 0, 1))
    z = z_vals[..., 0]
    out_t = pl.pallas_call(
        _evolve_block,
        grid=(B // _RBLK,),
        in_specs=[
            pl.BlockSpec((3, _RBLK, T), lambda i: (0, i, 0)),
            pl.BlockSpec((_RBLK, T), lambda i: (i, 0)),
            pl.BlockSpec((_RBLK, S), lambda i: (i, 0)),
        ],
        out_specs=pl.BlockSpec((3, _RBLK, S), lambda i: (0, i, 0)),
        out_shape=jax.ShapeDtypeStruct((3, B, S), jnp.float32),
    )(hist_t, distances, z)
    return jnp.transpose(out_t, (1, 2, 0))


# speedup vs baseline: 47.0792x; 1.0025x over previous
import jax
import jax.numpy as jnp
from jax.experimental import pallas as pl

_RBLK = 128


def _evolve_block(hist_ref, d_ref, z_ref, out_ref):
    z = z_ref[...]
    d = d_ref[...]
    key = z[:, :, None] - d[:, None, :]
    pos = jnp.where(key < 0.0, jnp.float32(10.0), key)
    neg = jnp.where(key > 0.0, jnp.float32(-10.0), key)
    vals = jnp.min(pos, axis=-1)
    idx0 = jnp.argmin(pos, axis=-1).astype(jnp.int32)
    idx1 = jnp.argmax(neg, axis=-1).astype(jnp.int32)
    idx = jnp.concatenate([idx0, idx1], axis=-1)
    g0 = [None] * 3
    g1 = [None] * 3
    S = z.shape[1]
    for c in range(3):
        hc = hist_ref[c]
        g = jnp.take_along_axis(hc, idx, axis=-1)
        g0[c] = g[:, :S]
        g1[c] = g[:, S:]
    m = [(g1[c] - g0[c]) / z for c in range(3)]
    norm = jnp.sqrt(m[0] * m[0] + m[1] * m[1] + m[2] * m[2])
    for c in range(3):
        out_ref[c] = g0[c] + vals * (m[c] / norm)


def kernel(r_hist, distances, z_vals):
    B, T = distances.shape
    S = z_vals.shape[1]
    hist_t = jnp.transpose(r_hist, (2, 0, 1))
    z = z_vals[..., 0]
    out_t = pl.pallas_call(
        _evolve_block,
        grid=(B // _RBLK,),
        in_specs=[
            pl.BlockSpec((3, _RBLK, T), lambda i: (0, i, 0)),
            pl.BlockSpec((_RBLK, T), lambda i: (i, 0)),
            pl.BlockSpec((_RBLK, S), lambda i: (i, 0)),
        ],
        out_specs=pl.BlockSpec((3, _RBLK, S), lambda i: (0, i, 0)),
        out_shape=jax.ShapeDtypeStruct((3, B, S), jnp.float32),
    )(hist_t, distances, z)
    return jnp.transpose(out_t, (1, 2, 0))


# speedup vs baseline: 94.9607x; 2.0170x over previous
import jax
import jax.numpy as jnp
from jax.experimental import pallas as pl

_RBLK = 64


def _evolve_block(hist_ref, d_ref, z_ref, out_ref):
    z = z_ref[...]
    d = d_ref[...]
    R, S = z.shape
    T = d.shape[1]
    L = 2 * S
    hx = hist_ref[0]
    hy = hist_ref[1]
    hz = hist_ref[2]
    zz = jnp.concatenate([z, z], axis=-1)
    lane = jax.lax.broadcasted_iota(jnp.int32, (R, L), 1)
    sgn = jnp.where(lane < S, jnp.float32(-1.0), jnp.float32(1.0))
    best = jnp.full((R, L), 10.0, dtype=jnp.float32)
    px = jnp.broadcast_to(hx[:, 0:1], (R, L))
    py = jnp.broadcast_to(hy[:, 0:1], (R, L))
    pz = jnp.broadcast_to(hz[:, 0:1], (R, L))
    for t in range(T):
        dcol = jnp.broadcast_to(d[:, t:t + 1], (R, L))
        u = (dcol - zz) * sgn
        cond = (u >= 0.0) & (u < best)
        best = jnp.where(cond, u, best)
        cx = jnp.broadcast_to(hx[:, t:t + 1], (R, L))
        cy = jnp.broadcast_to(hy[:, t:t + 1], (R, L))
        cz = jnp.broadcast_to(hz[:, t:t + 1], (R, L))
        px = jnp.where(cond, cx, px)
        py = jnp.where(cond, cy, py)
        pz = jnp.where(cond, cz, pz)
    vals = best[:, :S]
    mx = (px[:, S:] - px[:, :S]) / z
    my = (py[:, S:] - py[:, :S]) / z
    mz = (pz[:, S:] - pz[:, :S]) / z
    norm = jnp.sqrt(mx * mx + my * my + mz * mz)
    out_ref[0] = px[:, :S] + vals * (mx / norm)
    out_ref[1] = py[:, :S] + vals * (my / norm)
    out_ref[2] = pz[:, :S] + vals * (mz / norm)


def kernel(r_hist, distances, z_vals):
    B, T = distances.shape
    S = z_vals.shape[1]
    hist_t = jnp.transpose(r_hist, (2, 0, 1))
    z = z_vals[..., 0]
    out_t = pl.pallas_call(
        _evolve_block,
        grid=(B // _RBLK,),
        in_specs=[
            pl.BlockSpec((3, _RBLK, T), lambda i: (0, i, 0)),
            pl.BlockSpec((_RBLK, T), lambda i: (i, 0)),
            pl.BlockSpec((_RBLK, S), lambda i: (i, 0)),
        ],
        out_specs=pl.BlockSpec((3, _RBLK, S), lambda i: (0, i, 0)),
        out_shape=jax.ShapeDtypeStruct((3, B, S), jnp.float32),
    )(hist_t, distances, z)
    return jnp.transpose(out_t, (1, 2, 0))
